# Initial kernel scaffold; baseline (speedup 1.0000x reference)
import jax
import jax.numpy as jnp
from jax.experimental import pallas as pl


def kernel(x, edge_index, batch, W_emb, b_emb, W0, b0, gamma0, beta0, W1, b1, gamma1, beta1, W2, b2, gamma2, beta2):
    raise NotImplementedError("write your pallas kernel here")


# speedup vs baseline: 6.0534x; 6.0534x over previous
import jax
import jax.numpy as jnp
from jax import lax
from jax.experimental import pallas as pl
from jax.experimental.pallas import tpu as pltpu
from jax.experimental.pallas import tpu_sc as plsc

N = 10000
D = 128
H = 128
G = 64
EPS = 1e-5

NP = 10240
NC, NS = 2, 16
NW = NC * NS
CHUNK = 128
ROWS_PER_SUB = NP // NS
RB = 512


def _make_sc_agg(chunks_per_worker: int, with_deg: bool):
    mesh = plsc.VectorSubcoreMesh(core_axis_name="c", subcore_axis_name="s",
                                  num_cores=NC, num_subcores=NS)
    out_type = [jax.ShapeDtypeStruct((NC, NP, H), jnp.float32)]
    scratch = [
        pltpu.VMEM((1, CHUNK), jnp.int32),
        pltpu.VMEM((1, CHUNK), jnp.int32),
        pltpu.VMEM((CHUNK, H), jnp.float32),
        pltpu.VMEM_SHARED((NP, H), jnp.float32),
    ]
    if with_deg:
        out_type.append(jax.ShapeDtypeStruct((NW, NP), jnp.float32))
        scratch.append(pltpu.VMEM((NP,), jnp.float32))
    scratch.append(pltpu.SemaphoreType.DMA)

    def body(h_hbm, src_hbm, dst_hbm, z_hbm, *rest):
        if with_deg:
            (out_agg, out_deg, sidx, didx, stage, agg_sh, degloc, sem) = rest
        else:
            (out_agg, sidx, didx, stage, agg_sh, sem) = rest
        c = lax.axis_index("c")
        s = lax.axis_index("s")
        row0 = s * ROWS_PER_SUB
        wid = c * NS + s

        pltpu.sync_copy(z_hbm, stage)
        for k in range(ROWS_PER_SUB // CHUNK):
            pltpu.sync_copy(stage, agg_sh.at[pl.ds(row0 + k * CHUNK, CHUNK)])
        if with_deg:
            z16v = jnp.zeros((16,), jnp.float32)

            def zi(j, carry):
                degloc[pl.ds(j * 16, 16)] = z16v
                return carry

            lax.fori_loop(0, NP // 16, zi, 0)
        plsc.subcore_barrier()

        base_chunk = wid * chunks_per_worker
        if with_deg:
            ones16 = jnp.ones((16,), jnp.float32)

        def it(g, carry):
            base = (base_chunk + g) * CHUNK
            pltpu.sync_copy(src_hbm.at[pl.ds(base, CHUNK)], sidx.at[0])
            pltpu.sync_copy(dst_hbm.at[pl.ds(base, CHUNK)], didx.at[0])
            pltpu.async_copy(h_hbm.at[sidx.at[0]], stage, sem).wait()
            pltpu.sync_copy(stage, agg_sh.at[didx.at[0]], add=True)
            if with_deg:
                for j in range(CHUNK // 16):
                    idx = didx[0, pl.ds(j * 16, 16)]
                    plsc.addupdate_scatter(degloc, [idx], ones16)
            return carry

        lax.fori_loop(0, chunks_per_worker, it, 0)
        plsc.subcore_barrier()

        for k in range(ROWS_PER_SUB // CHUNK):
            r = row0 + k * CHUNK
            pltpu.sync_copy(agg_sh.at[pl.ds(r, CHUNK)], stage)
            pltpu.sync_copy(stage, out_agg.at[c, pl.ds(r, CHUNK)])
        if with_deg:
            pltpu.sync_copy(degloc, out_deg.at[wid])

    return pl.kernel(
        body, out_type=tuple(out_type), mesh=mesh,
        compiler_params=pltpu.CompilerParams(needs_layout_passes=False),
        scratch_types=tuple(scratch))


def _emb_body(x_ref, w_ref, b_ref, o_ref):
    t = jnp.dot(x_ref[...], w_ref[...], preferred_element_type=jnp.float32)
    o_ref[...] = jnp.maximum(t + b_ref[...], 0.0)


def _emb(x, w, b):
    return pl.pallas_call(
        _emb_body,
        grid=(NP // RB,),
        in_specs=[
            pl.BlockSpec((RB, D), lambda i: (i, 0)),
            pl.BlockSpec((D, H), lambda i: (0, 0)),
            pl.BlockSpec((1, H), lambda i: (0, 0)),
        ],
        out_specs=pl.BlockSpec((RB, H), lambda i: (i, 0)),
        out_shape=jax.ShapeDtypeStruct((NP, H), jnp.float32),
    )(x, w, b)


def _conv_mm0_body(p_ref, degp_ref, w_ref, b_ref, t_ref, stats_ref, deg_ref):
    i = pl.program_id(0)
    deg = jnp.sum(degp_ref[...], axis=0).reshape(RB, 1)
    deg_ref[...] = deg
    agg = (p_ref[0] + p_ref[1]) / jnp.maximum(deg, 1.0)
    t = jnp.dot(agg, w_ref[...], preferred_element_type=jnp.float32) + b_ref[...]
    t_ref[...] = t
    rows = i * RB + lax.broadcasted_iota(jnp.int32, (RB, 1), 0)
    tm = jnp.where(rows < N, t, 0.0)

    @pl.when(i == 0)
    def _():
        stats_ref[...] = jnp.zeros_like(stats_ref)

    stats_ref[0:1, :] += jnp.sum(tm, axis=0, keepdims=True)
    stats_ref[1:2, :] += jnp.sum(tm * tm, axis=0, keepdims=True)


def _conv_mm0(p, degp, w, b):
    return pl.pallas_call(
        _conv_mm0_body,
        grid=(NP // RB,),
        in_specs=[
            pl.BlockSpec((NC, RB, H), lambda i: (0, i, 0)),
            pl.BlockSpec((NW, RB), lambda i: (0, i)),
            pl.BlockSpec((H, H), lambda i: (0, 0)),
            pl.BlockSpec((1, H), lambda i: (0, 0)),
        ],
        out_specs=[
            pl.BlockSpec((RB, H), lambda i: (i, 0)),
            pl.BlockSpec((2, H), lambda i: (0, 0)),
            pl.BlockSpec((RB, 1), lambda i: (i, 0)),
        ],
        out_shape=[
            jax.ShapeDtypeStruct((NP, H), jnp.float32),
            jax.ShapeDtypeStruct((2, H), jnp.float32),
            jax.ShapeDtypeStruct((NP, 1), jnp.float32),
        ],
    )(p, degp, w, b)


def _conv_mm_body(p_ref, deg_ref, w_ref, b_ref, t_ref, stats_ref):
    i = pl.program_id(0)
    agg = (p_ref[0] + p_ref[1]) / jnp.maximum(deg_ref[...], 1.0)
    t = jnp.dot(agg, w_ref[...], preferred_element_type=jnp.float32) + b_ref[...]
    t_ref[...] = t
    rows = i * RB + lax.broadcasted_iota(jnp.int32, (RB, 1), 0)
    tm = jnp.where(rows < N, t, 0.0)

    @pl.when(i == 0)
    def _():
        stats_ref[...] = jnp.zeros_like(stats_ref)

    stats_ref[0:1, :] += jnp.sum(tm, axis=0, keepdims=True)
    stats_ref[1:2, :] += jnp.sum(tm * tm, axis=0, keepdims=True)


def _conv_mm(p, deg, w, b):
    return pl.pallas_call(
        _conv_mm_body,
        grid=(NP // RB,),
        in_specs=[
            pl.BlockSpec((NC, RB, H), lambda i: (0, i, 0)),
            pl.BlockSpec((RB, 1), lambda i: (i, 0)),
            pl.BlockSpec((H, H), lambda i: (0, 0)),
            pl.BlockSpec((1, H), lambda i: (0, 0)),
        ],
        out_specs=[
            pl.BlockSpec((RB, H), lambda i: (i, 0)),
            pl.BlockSpec((2, H), lambda i: (0, 0)),
        ],
        out_shape=[
            jax.ShapeDtypeStruct((NP, H), jnp.float32),
            jax.ShapeDtypeStruct((2, H), jnp.float32),
        ],
    )(p, deg, w, b)


def _bn_res_body(t_ref, stats_ref, g_ref, be_ref, h_ref, o_ref):
    mean = stats_ref[0:1, :] / N
    var = stats_ref[1:2, :] / N - mean * mean
    norm = (t_ref[...] - mean) / jnp.sqrt(var + EPS)
    o_ref[...] = h_ref[...] + jnp.maximum(norm * g_ref[...] + be_ref[...], 0.0)


def _bn_res(t, stats, gamma, beta, h):
    return pl.pallas_call(
        _bn_res_body,
        grid=(NP // RB,),
        in_specs=[
            pl.BlockSpec((RB, H), lambda i: (i, 0)),
            pl.BlockSpec((2, H), lambda i: (0, 0)),
            pl.BlockSpec((1, H), lambda i: (0, 0)),
            pl.BlockSpec((1, H), lambda i: (0, 0)),
            pl.BlockSpec((RB, H), lambda i: (i, 0)),
        ],
        out_specs=pl.BlockSpec((RB, H), lambda i: (i, 0)),
        out_shape=jax.ShapeDtypeStruct((NP, H), jnp.float32),
    )(t, stats, gamma, beta, h)


def _pool_body(b_ref, h_ref, o_ref, acc_ref, cnt_ref):
    i = pl.program_id(0)

    @pl.when(i == 0)
    def _():
        acc_ref[...] = jnp.zeros_like(acc_ref)
        cnt_ref[...] = jnp.zeros_like(cnt_ref)

    gids = lax.broadcasted_iota(jnp.int32, (G, RB), 0).astype(jnp.float32)
    m = (b_ref[...] == gids).astype(jnp.float32)
    acc_ref[...] += jnp.dot(m, h_ref[...], preferred_element_type=jnp.float32)
    cnt_ref[...] += jnp.sum(m, axis=1, keepdims=True)

    @pl.when(i == pl.num_programs(0) - 1)
    def _():
        o_ref[...] = acc_ref[...] / jnp.maximum(cnt_ref[...], 1.0)


def _pool(batchf, h):
    return pl.pallas_call(
        _pool_body,
        grid=(NP // RB,),
        in_specs=[
            pl.BlockSpec((1, RB), lambda i: (0, i)),
            pl.BlockSpec((RB, H), lambda i: (i, 0)),
        ],
        out_specs=pl.BlockSpec((G, H), lambda i: (0, 0)),
        out_shape=jax.ShapeDtypeStruct((G, H), jnp.float32),
        scratch_shapes=[
            pltpu.VMEM((G, H), jnp.float32),
            pltpu.VMEM((G, 1), jnp.float32),
        ],
    )(batchf, h)


def kernel(x, edge_index, batch, W_emb, b_emb,
           W0, b0, gamma0, beta0,
           W1, b1, gamma1, beta1,
           W2, b2, gamma2, beta2):
    E = edge_index.shape[1]
    chunks_per_worker = -(-E // (CHUNK * NW))
    epad = chunks_per_worker * CHUNK * NW
    npad_e = epad - E

    src = edge_index[0].astype(jnp.int32)
    dst = edge_index[1].astype(jnp.int32)
    pad_ids = jnp.arange(npad_e, dtype=jnp.int32)
    src = jnp.concatenate([src, pad_ids % N])
    dst = jnp.concatenate([dst, N + pad_ids % (NP - N)])

    xp = jnp.zeros((NP, D), jnp.float32).at[:N].set(x.astype(jnp.float32))
    batchf = jnp.full((1, NP), float(G), jnp.float32).at[0, :N].set(
        batch.astype(jnp.float32))
    z128 = jnp.zeros((CHUNK, H), jnp.float32)

    sc_deg = _make_sc_agg(chunks_per_worker, True)
    sc_nodeg = _make_sc_agg(chunks_per_worker, False)

    h = _emb(xp, W_emb.astype(jnp.float32), b_emb.reshape(1, H))

    deg = None
    for li, (W, b, gm, be) in enumerate(
            [(W0, b0, gamma0, beta0), (W1, b1, gamma1, beta1),
             (W2, b2, gamma2, beta2)]):
        if li == 0:
            p, degp = sc_deg(h, src, dst, z128)
            t, stats, deg = _conv_mm0(p, degp, W, b.reshape(1, H))
        else:
            (p,) = sc_nodeg(h, src, dst, z128)
            t, stats = _conv_mm(p, deg, W, b.reshape(1, H))
        h = _bn_res(t, stats, gm.reshape(1, H), be.reshape(1, H), h)

    return _pool(batchf, h)


# speedup vs baseline: 7.0878x; 1.1709x over previous
import jax
import jax.numpy as jnp
from jax import lax
from jax.experimental import pallas as pl
from jax.experimental.pallas import tpu as pltpu
from jax.experimental.pallas import tpu_sc as plsc

N = 10000
D = 128
H = 128
G = 64
EPS = 1e-5

NP = 10240
NC, NS = 2, 16
NW = NC * NS
CHUNK = 128
ROWS_PER_SUB = NP // NS
RB = 512


NBUF = 2


def _make_sc_agg(chunks_per_worker: int, with_deg: bool):
    mesh = plsc.VectorSubcoreMesh(core_axis_name="c", subcore_axis_name="s",
                                  num_cores=NC, num_subcores=NS)
    cpw = chunks_per_worker
    out_type = [jax.ShapeDtypeStruct((NC, NP, H), jnp.float32)]
    scratch = [pltpu.VMEM((1, CHUNK), jnp.int32) for _ in range(2 * NBUF)]
    scratch += [pltpu.VMEM((CHUNK, H), jnp.float32) for _ in range(NBUF)]
    scratch.append(pltpu.VMEM_SHARED((NP, H), jnp.float32))
    if with_deg:
        out_type.append(jax.ShapeDtypeStruct((NW, NP), jnp.float32))
        scratch.append(pltpu.VMEM((NP,), jnp.float32))
    scratch += [pltpu.SemaphoreType.DMA for _ in range(2 * NBUF)]

    def body(h_hbm, src_hbm, dst_hbm, z_hbm, *rest):
        if with_deg:
            (out_agg, out_deg, *rest2) = rest
        else:
            (out_agg, *rest2) = rest
        sidx = rest2[:NBUF]
        didx = rest2[NBUF:2 * NBUF]
        st = rest2[2 * NBUF:3 * NBUF]
        agg_sh = rest2[3 * NBUF]
        if with_deg:
            degloc = rest2[3 * NBUF + 1]
            sems = rest2[3 * NBUF + 2:]
        else:
            sems = rest2[3 * NBUF + 1:]
        gsem, ssem = sems[:NBUF], sems[NBUF:]
        c = lax.axis_index("c")
        s = lax.axis_index("s")
        row0 = s * ROWS_PER_SUB
        wid = c * NS + s

        pltpu.sync_copy(z_hbm, st[0])
        for k in range(ROWS_PER_SUB // CHUNK):
            pltpu.sync_copy(st[0], agg_sh.at[pl.ds(row0 + k * CHUNK, CHUNK)])
        if with_deg:
            z16v = jnp.zeros((16,), jnp.float32)

            def zi(j, carry):
                degloc[pl.ds(j * 16, 16)] = z16v
                return carry

            lax.fori_loop(0, NP // 16, zi, 0)
        plsc.subcore_barrier()

        if with_deg:
            ones16 = jnp.ones((16,), jnp.float32)
        base_chunk = wid * cpw

        def wait_gather(b):
            pltpu.make_async_copy(h_hbm.at[pl.ds(0, CHUNK)], st[b],
                                  gsem[b]).wait()

        def wait_scatter(b):
            pltpu.make_async_copy(st[b], agg_sh.at[pl.ds(0, CHUNK)],
                                  ssem[b]).wait()

        def load_idx(g, b):
            pltpu.sync_copy(src_hbm.at[base_chunk + g], sidx[b].at[0])
            pltpu.sync_copy(dst_hbm.at[base_chunk + g], didx[b].at[0])

        def start_gather(b):
            pltpu.async_copy(h_hbm.at[sidx[b].at[0]], st[b], gsem[b])

        def section(g, b, prime, prefetch):
            o = b ^ 1
            wait_gather(b)
            pltpu.async_copy(st[b], agg_sh.at[didx[b].at[0]], ssem[b],
                             add=True)
            if with_deg:
                for j in range(CHUNK // 16):
                    idx = didx[b][0, pl.ds(j * 16, 16)]
                    plsc.addupdate_scatter(degloc, [idx], ones16)
            if prefetch:
                if not prime:
                    wait_scatter(o)
                load_idx(g + 1, o)
                start_gather(o)

        load_idx(0, 0)
        start_gather(0)
        section(0, 0, True, True)
        section(1, 1, False, True)

        def oiter(oo, carry):
            section(2 * oo, 0, False, True)
            section(2 * oo + 1, 1, False, True)
            return carry

        lax.fori_loop(1, cpw // 2 - 1, oiter, 0)
        section(cpw - 2, 0, False, True)
        section(cpw - 1, 1, False, False)
        for b in range(NBUF):
            wait_scatter(b)
        plsc.subcore_barrier()

        for k in range(ROWS_PER_SUB // CHUNK):
            r = row0 + k * CHUNK
            b = k % NBUF
            if k >= NBUF:
                pltpu.make_async_copy(st[b], out_agg.at[c, pl.ds(row0, CHUNK)],
                                      ssem[b]).wait()
            pltpu.sync_copy(agg_sh.at[pl.ds(r, CHUNK)], st[b])
            pltpu.async_copy(st[b], out_agg.at[c, pl.ds(r, CHUNK)], ssem[b])
        for k in range(min(NBUF, ROWS_PER_SUB // CHUNK)):
            pltpu.make_async_copy(st[k], out_agg.at[c, pl.ds(row0, CHUNK)],
                                  ssem[k]).wait()
        if with_deg:
            pltpu.sync_copy(degloc, out_deg.at[wid])

    return pl.kernel(
        body, out_type=tuple(out_type), mesh=mesh,
        compiler_params=pltpu.CompilerParams(needs_layout_passes=False),
        scratch_types=tuple(scratch))


def _emb_body(x_ref, w_ref, b_ref, o_ref):
    t = jnp.dot(x_ref[...], w_ref[...], preferred_element_type=jnp.float32)
    o_ref[...] = jnp.maximum(t + b_ref[...], 0.0)


def _emb(x, w, b):
    return pl.pallas_call(
        _emb_body,
        grid=(NP // RB,),
        in_specs=[
            pl.BlockSpec((RB, D), lambda i: (i, 0)),
            pl.BlockSpec((D, H), lambda i: (0, 0)),
            pl.BlockSpec((1, H), lambda i: (0, 0)),
        ],
        out_specs=pl.BlockSpec((RB, H), lambda i: (i, 0)),
        out_shape=jax.ShapeDtypeStruct((NP, H), jnp.float32),
    )(x, w, b)


def _conv_mm0_body(p_ref, degp_ref, w_ref, b_ref, t_ref, stats_ref, deg_ref):
    i = pl.program_id(0)
    deg = jnp.sum(degp_ref[...], axis=0).reshape(RB, 1)
    deg_ref[...] = deg
    agg = (p_ref[0] + p_ref[1]) / jnp.maximum(deg, 1.0)
    t = jnp.dot(agg, w_ref[...], preferred_element_type=jnp.float32) + b_ref[...]
    t_ref[...] = t
    rows = i * RB + lax.broadcasted_iota(jnp.int32, (RB, 1), 0)
    tm = jnp.where(rows < N, t, 0.0)

    @pl.when(i == 0)
    def _():
        stats_ref[...] = jnp.zeros_like(stats_ref)

    stats_ref[0:1, :] += jnp.sum(tm, axis=0, keepdims=True)
    stats_ref[1:2, :] += jnp.sum(tm * tm, axis=0, keepdims=True)


def _conv_mm0(p, degp, w, b):
    return pl.pallas_call(
        _conv_mm0_body,
        grid=(NP // RB,),
        in_specs=[
            pl.BlockSpec((NC, RB, H), lambda i: (0, i, 0)),
            pl.BlockSpec((NW, RB), lambda i: (0, i)),
            pl.BlockSpec((H, H), lambda i: (0, 0)),
            pl.BlockSpec((1, H), lambda i: (0, 0)),
        ],
        out_specs=[
            pl.BlockSpec((RB, H), lambda i: (i, 0)),
            pl.BlockSpec((2, H), lambda i: (0, 0)),
            pl.BlockSpec((RB, 1), lambda i: (i, 0)),
        ],
        out_shape=[
            jax.ShapeDtypeStruct((NP, H), jnp.float32),
            jax.ShapeDtypeStruct((2, H), jnp.float32),
            jax.ShapeDtypeStruct((NP, 1), jnp.float32),
        ],
    )(p, degp, w, b)


def _conv_mm_body(p_ref, deg_ref, w_ref, b_ref, t_ref, stats_ref):
    i = pl.program_id(0)
    agg = (p_ref[0] + p_ref[1]) / jnp.maximum(deg_ref[...], 1.0)
    t = jnp.dot(agg, w_ref[...], preferred_element_type=jnp.float32) + b_ref[...]
    t_ref[...] = t
    rows = i * RB + lax.broadcasted_iota(jnp.int32, (RB, 1), 0)
    tm = jnp.where(rows < N, t, 0.0)

    @pl.when(i == 0)
    def _():
        stats_ref[...] = jnp.zeros_like(stats_ref)

    stats_ref[0:1, :] += jnp.sum(tm, axis=0, keepdims=True)
    stats_ref[1:2, :] += jnp.sum(tm * tm, axis=0, keepdims=True)


def _conv_mm(p, deg, w, b):
    return pl.pallas_call(
        _conv_mm_body,
        grid=(NP // RB,),
        in_specs=[
            pl.BlockSpec((NC, RB, H), lambda i: (0, i, 0)),
            pl.BlockSpec((RB, 1), lambda i: (i, 0)),
            pl.BlockSpec((H, H), lambda i: (0, 0)),
            pl.BlockSpec((1, H), lambda i: (0, 0)),
        ],
        out_specs=[
            pl.BlockSpec((RB, H), lambda i: (i, 0)),
            pl.BlockSpec((2, H), lambda i: (0, 0)),
        ],
        out_shape=[
            jax.ShapeDtypeStruct((NP, H), jnp.float32),
            jax.ShapeDtypeStruct((2, H), jnp.float32),
        ],
    )(p, deg, w, b)


def _bn_res_body(t_ref, stats_ref, g_ref, be_ref, h_ref, o_ref):
    mean = stats_ref[0:1, :] / N
    var = stats_ref[1:2, :] / N - mean * mean
    norm = (t_ref[...] - mean) / jnp.sqrt(var + EPS)
    o_ref[...] = h_ref[...] + jnp.maximum(norm * g_ref[...] + be_ref[...], 0.0)


def _bn_res(t, stats, gamma, beta, h):
    return pl.pallas_call(
        _bn_res_body,
        grid=(NP // RB,),
        in_specs=[
            pl.BlockSpec((RB, H), lambda i: (i, 0)),
            pl.BlockSpec((2, H), lambda i: (0, 0)),
            pl.BlockSpec((1, H), lambda i: (0, 0)),
            pl.BlockSpec((1, H), lambda i: (0, 0)),
            pl.BlockSpec((RB, H), lambda i: (i, 0)),
        ],
        out_specs=pl.BlockSpec((RB, H), lambda i: (i, 0)),
        out_shape=jax.ShapeDtypeStruct((NP, H), jnp.float32),
    )(t, stats, gamma, beta, h)


def _pool_body(b_ref, h_ref, o_ref, acc_ref, cnt_ref):
    i = pl.program_id(0)

    @pl.when(i == 0)
    def _():
        acc_ref[...] = jnp.zeros_like(acc_ref)
        cnt_ref[...] = jnp.zeros_like(cnt_ref)

    gids = lax.broadcasted_iota(jnp.int32, (G, RB), 0).astype(jnp.float32)
    m = (b_ref[...] == gids).astype(jnp.float32)
    acc_ref[...] += jnp.dot(m, h_ref[...], preferred_element_type=jnp.float32)
    cnt_ref[...] += jnp.sum(m, axis=1, keepdims=True)

    @pl.when(i == pl.num_programs(0) - 1)
    def _():
        o_ref[...] = acc_ref[...] / jnp.maximum(cnt_ref[...], 1.0)


def _pool(batchf, h):
    return pl.pallas_call(
        _pool_body,
        grid=(NP // RB,),
        in_specs=[
            pl.BlockSpec((1, RB), lambda i: (0, i)),
            pl.BlockSpec((RB, H), lambda i: (i, 0)),
        ],
        out_specs=pl.BlockSpec((G, H), lambda i: (0, 0)),
        out_shape=jax.ShapeDtypeStruct((G, H), jnp.float32),
        scratch_shapes=[
            pltpu.VMEM((G, H), jnp.float32),
            pltpu.VMEM((G, 1), jnp.float32),
        ],
    )(batchf, h)


def kernel(x, edge_index, batch, W_emb, b_emb,
           W0, b0, gamma0, beta0,
           W1, b1, gamma1, beta1,
           W2, b2, gamma2, beta2):
    E = edge_index.shape[1]
    chunks_per_worker = -(-E // (CHUNK * NW))
    chunks_per_worker = -(-chunks_per_worker // NBUF) * NBUF
    epad = chunks_per_worker * CHUNK * NW
    npad_e = epad - E

    src = edge_index[0].astype(jnp.int32)
    dst = edge_index[1].astype(jnp.int32)
    pad_ids = jnp.arange(npad_e, dtype=jnp.int32)
    src = jnp.concatenate([src, pad_ids % N]).reshape(-1, CHUNK)
    dst = jnp.concatenate([dst, N + pad_ids % (NP - N)]).reshape(-1, CHUNK)

    xp = jnp.zeros((NP, D), jnp.float32).at[:N].set(x.astype(jnp.float32))
    batchf = jnp.full((1, NP), float(G), jnp.float32).at[0, :N].set(
        batch.astype(jnp.float32))
    z128 = jnp.zeros((CHUNK, H), jnp.float32)

    sc_deg = _make_sc_agg(chunks_per_worker, True)
    sc_nodeg = _make_sc_agg(chunks_per_worker, False)

    h = _emb(xp, W_emb.astype(jnp.float32), b_emb.reshape(1, H))

    deg = None
    for li, (W, b, gm, be) in enumerate(
            [(W0, b0, gamma0, beta0), (W1, b1, gamma1, beta1),
             (W2, b2, gamma2, beta2)]):
        if li == 0:
            p, degp = sc_deg(h, src, dst, z128)
            t, stats, deg = _conv_mm0(p, degp, W, b.reshape(1, H))
        else:
            p, _unused = sc_deg(h, src, dst, z128)
            t, stats = _conv_mm(p, deg, W, b.reshape(1, H))
        h = _bn_res(t, stats, gm.reshape(1, H), be.reshape(1, H), h)

    return _pool(batchf, h)


# speedup vs baseline: 9.9079x; 1.3979x over previous
import jax
import jax.numpy as jnp
from jax import lax
from jax.experimental import pallas as pl
from jax.experimental.pallas import tpu as pltpu
from jax.experimental.pallas import tpu_sc as plsc

N = 10000
D = 128
H = 128
G = 64
EPS = 1e-5

NP = 10240
NC, NS = 2, 16
NW = NC * NS
CHUNK = 128
ROWS_PER_SUB = NP // NS
RB = 512


NBUF = 2


def _make_sc_agg(chunks_per_worker: int, with_deg: bool):
    mesh = plsc.VectorSubcoreMesh(core_axis_name="c", subcore_axis_name="s",
                                  num_cores=NC, num_subcores=NS)
    cpw = chunks_per_worker
    NI = 4
    out_type = [jax.ShapeDtypeStruct((NC, NP, H), jnp.float32)]
    scratch = [pltpu.VMEM((2, CHUNK), jnp.int32) for _ in range(NI)]
    scratch += [pltpu.VMEM((CHUNK, H), jnp.float32) for _ in range(NBUF)]
    scratch.append(pltpu.VMEM_SHARED((NP, H), jnp.float32))
    if with_deg:
        out_type.append(jax.ShapeDtypeStruct((NW, NP), jnp.float32))
        scratch.append(pltpu.VMEM((NP,), jnp.float32))
    scratch += [pltpu.SemaphoreType.DMA for _ in range(2 * NBUF + NI)]

    def body(h_hbm, ed_hbm, z_hbm, *rest):
        if with_deg:
            (out_agg, out_deg, *rest2) = rest
        else:
            (out_agg, *rest2) = rest
        idxr = rest2[:NI]
        st = rest2[NI:NI + NBUF]
        agg_sh = rest2[NI + NBUF]
        if with_deg:
            degloc = rest2[NI + NBUF + 1]
            sems = rest2[NI + NBUF + 2:]
        else:
            sems = rest2[NI + NBUF + 1:]
        gsem, ssem = sems[:NBUF], sems[NBUF:2 * NBUF]
        isem = sems[2 * NBUF:]
        c = lax.axis_index("c")
        s = lax.axis_index("s")
        row0 = s * ROWS_PER_SUB
        wid = c * NS + s

        pltpu.sync_copy(z_hbm, st[0])
        for k in range(ROWS_PER_SUB // CHUNK):
            pltpu.sync_copy(st[0], agg_sh.at[pl.ds(row0 + k * CHUNK, CHUNK)])
        if with_deg:
            z16v = jnp.zeros((16,), jnp.float32)

            def zi(j, carry):
                degloc[pl.ds(j * 16, 16)] = z16v
                return carry

            lax.fori_loop(0, NP // 16, zi, 0)
        plsc.subcore_barrier()

        if with_deg:
            ones16 = jnp.ones((16,), jnp.float32)
        base_chunk = wid * cpw

        def wait_gather(b):
            pltpu.make_async_copy(h_hbm.at[pl.ds(0, CHUNK)], st[b],
                                  gsem[b]).wait()

        def wait_scatter(b):
            pltpu.make_async_copy(st[b], agg_sh.at[pl.ds(0, CHUNK)],
                                  ssem[b]).wait()

        def load_idx(g, i):
            pltpu.async_copy(ed_hbm.at[base_chunk + g], idxr[i], isem[i])

        def wait_idx(i):
            pltpu.make_async_copy(ed_hbm.at[0], idxr[i], isem[i]).wait()

        def start_gather(b, i):
            pltpu.async_copy(h_hbm.at[idxr[i].at[0]], st[b], gsem[b])

        def section(g, b, i, prime, pf_idx, pf_gather):
            o = b ^ 1
            wait_gather(b)
            pltpu.async_copy(st[b], agg_sh.at[idxr[i].at[1]], ssem[b],
                             add=True)
            if with_deg:
                for j in range(CHUNK // 16):
                    idx = idxr[i][1, pl.ds(j * 16, 16)]
                    plsc.addupdate_scatter(degloc, [idx], ones16)
            if pf_gather:
                if not prime:
                    wait_scatter(o)
                if pf_idx:
                    load_idx(g + 3, (i + 3) % NI)
                wait_idx((i + 1) % NI)
                start_gather(o, (i + 1) % NI)

        for i in range(NI):
            load_idx(i, i)
        wait_idx(0)
        start_gather(0, 0)
        for i in range(NI):
            section(i, i % 2, i, i == 0, i >= 1, True)

        def oiter(oo, carry):
            for i in range(NI):
                section(NI * oo + i, i % 2, i, False, True, True)
            return carry

        lax.fori_loop(1, cpw // NI - 1, oiter, 0)
        g_last = cpw - NI
        for i in range(NI):
            section(g_last + i, i % 2, i, False, i + 3 < NI, i + 1 < NI)
        for b in range(NBUF):
            wait_scatter(b)
        plsc.subcore_barrier()

        for k in range(ROWS_PER_SUB // CHUNK):
            r = row0 + k * CHUNK
            b = k % NBUF
            if k >= NBUF:
                pltpu.make_async_copy(st[b], out_agg.at[c, pl.ds(row0, CHUNK)],
                                      ssem[b]).wait()
            pltpu.sync_copy(agg_sh.at[pl.ds(r, CHUNK)], st[b])
            pltpu.async_copy(st[b], out_agg.at[c, pl.ds(r, CHUNK)], ssem[b])
        for k in range(min(NBUF, ROWS_PER_SUB // CHUNK)):
            pltpu.make_async_copy(st[k], out_agg.at[c, pl.ds(row0, CHUNK)],
                                  ssem[k]).wait()
        if with_deg:
            pltpu.sync_copy(degloc, out_deg.at[wid])

    return pl.kernel(
        body, out_type=tuple(out_type), mesh=mesh,
        compiler_params=pltpu.CompilerParams(needs_layout_passes=False),
        scratch_types=tuple(scratch))


def _emb_body(x_ref, w_ref, b_ref, o_ref):
    t = jnp.dot(x_ref[...], w_ref[...], preferred_element_type=jnp.float32)
    o_ref[...] = jnp.maximum(t + b_ref[...], 0.0)


def _emb(x, w, b):
    return pl.pallas_call(
        _emb_body,
        grid=(NP // RB,),
        in_specs=[
            pl.BlockSpec((RB, D), lambda i: (i, 0)),
            pl.BlockSpec((D, H), lambda i: (0, 0)),
            pl.BlockSpec((1, H), lambda i: (0, 0)),
        ],
        out_specs=pl.BlockSpec((RB, H), lambda i: (i, 0)),
        out_shape=jax.ShapeDtypeStruct((NP, H), jnp.float32),
    )(x, w, b)


def _conv_mm0_body(p_ref, degp_ref, w_ref, b_ref, t_ref, stats_ref, deg_ref):
    i = pl.program_id(0)
    deg = jnp.sum(degp_ref[...], axis=0).reshape(RB, 1)
    deg_ref[...] = deg
    agg = (p_ref[0] + p_ref[1]) / jnp.maximum(deg, 1.0)
    t = jnp.dot(agg, w_ref[...], preferred_element_type=jnp.float32) + b_ref[...]
    t_ref[...] = t
    rows = i * RB + lax.broadcasted_iota(jnp.int32, (RB, 1), 0)
    tm = jnp.where(rows < N, t, 0.0)

    @pl.when(i == 0)
    def _():
        stats_ref[...] = jnp.zeros_like(stats_ref)

    stats_ref[0:1, :] += jnp.sum(tm, axis=0, keepdims=True)
    stats_ref[1:2, :] += jnp.sum(tm * tm, axis=0, keepdims=True)


def _conv_mm0(p, degp, w, b):
    return pl.pallas_call(
        _conv_mm0_body,
        grid=(NP // RB,),
        in_specs=[
            pl.BlockSpec((NC, RB, H), lambda i: (0, i, 0)),
            pl.BlockSpec((NW, RB), lambda i: (0, i)),
            pl.BlockSpec((H, H), lambda i: (0, 0)),
            pl.BlockSpec((1, H), lambda i: (0, 0)),
        ],
        out_specs=[
            pl.BlockSpec((RB, H), lambda i: (i, 0)),
            pl.BlockSpec((2, H), lambda i: (0, 0)),
            pl.BlockSpec((RB, 1), lambda i: (i, 0)),
        ],
        out_shape=[
            jax.ShapeDtypeStruct((NP, H), jnp.float32),
            jax.ShapeDtypeStruct((2, H), jnp.float32),
            jax.ShapeDtypeStruct((NP, 1), jnp.float32),
        ],
    )(p, degp, w, b)


def _conv_mm_body(p_ref, deg_ref, w_ref, b_ref, t_ref, stats_ref):
    i = pl.program_id(0)
    agg = (p_ref[0] + p_ref[1]) / jnp.maximum(deg_ref[...], 1.0)
    t = jnp.dot(agg, w_ref[...], preferred_element_type=jnp.float32) + b_ref[...]
    t_ref[...] = t
    rows = i * RB + lax.broadcasted_iota(jnp.int32, (RB, 1), 0)
    tm = jnp.where(rows < N, t, 0.0)

    @pl.when(i == 0)
    def _():
        stats_ref[...] = jnp.zeros_like(stats_ref)

    stats_ref[0:1, :] += jnp.sum(tm, axis=0, keepdims=True)
    stats_ref[1:2, :] += jnp.sum(tm * tm, axis=0, keepdims=True)


def _conv_mm(p, deg, w, b):
    return pl.pallas_call(
        _conv_mm_body,
        grid=(NP // RB,),
        in_specs=[
            pl.BlockSpec((NC, RB, H), lambda i: (0, i, 0)),
            pl.BlockSpec((RB, 1), lambda i: (i, 0)),
            pl.BlockSpec((H, H), lambda i: (0, 0)),
            pl.BlockSpec((1, H), lambda i: (0, 0)),
        ],
        out_specs=[
            pl.BlockSpec((RB, H), lambda i: (i, 0)),
            pl.BlockSpec((2, H), lambda i: (0, 0)),
        ],
        out_shape=[
            jax.ShapeDtypeStruct((NP, H), jnp.float32),
            jax.ShapeDtypeStruct((2, H), jnp.float32),
        ],
    )(p, deg, w, b)


def _bn_res_body(t_ref, stats_ref, g_ref, be_ref, h_ref, o_ref):
    mean = stats_ref[0:1, :] / N
    var = stats_ref[1:2, :] / N - mean * mean
    norm = (t_ref[...] - mean) / jnp.sqrt(var + EPS)
    o_ref[...] = h_ref[...] + jnp.maximum(norm * g_ref[...] + be_ref[...], 0.0)


def _bn_res(t, stats, gamma, beta, h):
    return pl.pallas_call(
        _bn_res_body,
        grid=(NP // RB,),
        in_specs=[
            pl.BlockSpec((RB, H), lambda i: (i, 0)),
            pl.BlockSpec((2, H), lambda i: (0, 0)),
            pl.BlockSpec((1, H), lambda i: (0, 0)),
            pl.BlockSpec((1, H), lambda i: (0, 0)),
            pl.BlockSpec((RB, H), lambda i: (i, 0)),
        ],
        out_specs=pl.BlockSpec((RB, H), lambda i: (i, 0)),
        out_shape=jax.ShapeDtypeStruct((NP, H), jnp.float32),
    )(t, stats, gamma, beta, h)


def _pool_body(b_ref, h_ref, o_ref, acc_ref, cnt_ref):
    i = pl.program_id(0)

    @pl.when(i == 0)
    def _():
        acc_ref[...] = jnp.zeros_like(acc_ref)
        cnt_ref[...] = jnp.zeros_like(cnt_ref)

    gids = lax.broadcasted_iota(jnp.int32, (G, RB), 0).astype(jnp.float32)
    m = (b_ref[...] == gids).astype(jnp.float32)
    acc_ref[...] += jnp.dot(m, h_ref[...], preferred_element_type=jnp.float32)
    cnt_ref[...] += jnp.sum(m, axis=1, keepdims=True)

    @pl.when(i == pl.num_programs(0) - 1)
    def _():
        o_ref[...] = acc_ref[...] / jnp.maximum(cnt_ref[...], 1.0)


def _pool(batchf, h):
    return pl.pallas_call(
        _pool_body,
        grid=(NP // RB,),
        in_specs=[
            pl.BlockSpec((1, RB), lambda i: (0, i)),
            pl.BlockSpec((RB, H), lambda i: (i, 0)),
        ],
        out_specs=pl.BlockSpec((G, H), lambda i: (0, 0)),
        out_shape=jax.ShapeDtypeStruct((G, H), jnp.float32),
        scratch_shapes=[
            pltpu.VMEM((G, H), jnp.float32),
            pltpu.VMEM((G, 1), jnp.float32),
        ],
    )(batchf, h)


def kernel(x, edge_index, batch, W_emb, b_emb,
           W0, b0, gamma0, beta0,
           W1, b1, gamma1, beta1,
           W2, b2, gamma2, beta2):
    E = edge_index.shape[1]
    chunks_per_worker = -(-E // (CHUNK * NW))
    chunks_per_worker = -(-chunks_per_worker // 4) * 4
    epad = chunks_per_worker * CHUNK * NW
    npad_e = epad - E

    src = edge_index[0].astype(jnp.int32)
    dst = edge_index[1].astype(jnp.int32)
    pad_ids = jnp.arange(npad_e, dtype=jnp.int32)
    src = jnp.concatenate([src, pad_ids % N]).reshape(-1, 1, CHUNK)
    dst = jnp.concatenate([dst, N + pad_ids % (NP - N)]).reshape(-1, 1, CHUNK)
    ed = jnp.concatenate([src, dst], axis=1)

    xp = jnp.zeros((NP, D), jnp.float32).at[:N].set(x.astype(jnp.float32))
    batchf = jnp.full((1, NP), float(G), jnp.float32).at[0, :N].set(
        batch.astype(jnp.float32))
    z128 = jnp.zeros((CHUNK, H), jnp.float32)

    sc_deg = _make_sc_agg(chunks_per_worker, True)
    sc_nodeg = _make_sc_agg(chunks_per_worker, False)

    h = _emb(xp, W_emb.astype(jnp.float32), b_emb.reshape(1, H))

    deg = None
    for li, (W, b, gm, be) in enumerate(
            [(W0, b0, gamma0, beta0), (W1, b1, gamma1, beta1),
             (W2, b2, gamma2, beta2)]):
        if li == 0:
            p, degp = sc_deg(h, ed, z128)
            t, stats, deg = _conv_mm0(p, degp, W, b.reshape(1, H))
        else:
            (p,) = sc_nodeg(h, ed, z128)
            t, stats = _conv_mm(p, deg, W, b.reshape(1, H))
        h = _bn_res(t, stats, gm.reshape(1, H), be.reshape(1, H), h)

    return _pool(batchf, h)
